# baseline (speedup 1.0000x reference)
import jax
import jax.numpy as jnp
from jax.experimental import pallas as pl

_NC = 25
_B = 1024
_S = 1024


_SPLIT = 4


def _onehot_body(idx_ref, out_ref):
    c = pl.program_id(0)
    idx = idx_ref[...]
    out_ref[0] = (idx == c).astype(jnp.float32)


def kernel(inputs):
    bm = _B // _SPLIT
    y = pl.pallas_call(
        _onehot_body,
        grid=(_NC, _SPLIT),
        in_specs=[pl.BlockSpec((bm, _S), lambda c, i: (i, 0))],
        out_specs=pl.BlockSpec((1, bm, _S), lambda c, i: (c, i, 0)),
        out_shape=jax.ShapeDtypeStruct((_NC, _B, _S), jnp.float32),
    )(inputs)
    return jnp.transpose(y, (1, 2, 0))


# speedup vs baseline: 1.7398x; 1.7398x over previous
import jax
import jax.numpy as jnp
from jax.experimental import pallas as pl

_NC = 25
_B = 1024
_S = 1024


_SPLIT = 4


def _onehot_body(idx_ref, out_ref):
    c = pl.program_id(1)
    idx = idx_ref[...]
    out_ref[0] = (idx == c).astype(jnp.float32)


def kernel(inputs):
    bm = _B // _SPLIT
    y = pl.pallas_call(
        _onehot_body,
        grid=(_SPLIT, _NC),
        in_specs=[pl.BlockSpec((bm, _S), lambda i, c: (i, 0))],
        out_specs=pl.BlockSpec((1, bm, _S), lambda i, c: (c, i, 0)),
        out_shape=jax.ShapeDtypeStruct((_NC, _B, _S), jnp.float32),
    )(inputs)
    return jnp.transpose(y, (1, 2, 0))
